# baseline (speedup 1.0000x reference)
import jax
import jax.numpy as jnp
from jax import lax
from jax.experimental import pallas as pl
from jax.experimental.pallas import tpu as pltpu
from jax.experimental.pallas import tpu_sc as plsc

B = 16384
D = 16
L = 50
NE = 8
F = (NE + 1) * D
NC = 2
NS = 16
NW = NC * NS
S = B // NW
C = 128
NCH = S // C
LP = 56

HIST_ROW = NE * D


def _sc_gather_body(idx8_hbm, hist_hbm,
                    t0, t1, t2, t3, t4, t5, t6, t7, ehT, dummy_hbm,
                    featsT_hbm,
                    idx8_v, histp_v, hi8_v, off8_v, hhi_v, hoff_v,
                    pbig_v, featsT_v,
                    sem0, sem1):
    tables = (t0, t1, t2, t3, t4, t5, t6, t7)
    sems = (sem0, sem1)
    wid = lax.axis_index("s") * NC + lax.axis_index("c")
    lane = lax.broadcasted_iota(jnp.int32, (16,), 0)

    def hi_ref(p):
        if isinstance(p, int) and p < NE:
            return hi8_v.at[p]
        return hhi_v.at[p - NE]

    def off_ref(p):
        if isinstance(p, int) and p < NE:
            return off8_v.at[p]
        return hoff_v.at[p - NE]

    def fire(p, src, b):
        idx = hi_ref(p)
        for d in range(D):
            pltpu.async_copy(src.at[d].at[idx], pbig_v.at[b, d], sems[b])

    def drain(b):
        pltpu.make_async_copy(dummy_hbm, pbig_v.at[b], sems[b]).wait()

    def select(p, frow, accumulate, b):
        offs = off_ref(p)
        for g in range(C // 16):
            rows = g * 16 + lane
            off16 = offs[pl.ds(g * 16, 16)]
            for d in range(D):
                v = plsc.load_gather(pbig_v.at[b, d], [rows, off16])
                dst = featsT_v.at[frow + d, pl.ds(g * 16, 16)]
                if accumulate:
                    plsc.addupdate(dst, v)
                else:
                    featsT_v[frow + d, pl.ds(g * 16, 16)] = v

    @pl.loop(0, NCH)
    def _chunk(c):
        base = wid * S + c * C

        pltpu.sync_copy(idx8_hbm.at[:, pl.ds(base, C)], idx8_v)
        pltpu.sync_copy(hist_hbm.at[:, pl.ds(base, C)], histp_v)

        for t in range(NE):
            @pl.loop(0, C // 16)
            def _sp8(g, t=t):
                raw = idx8_v[t, pl.ds(g * 16, 16)]
                hi8_v[t, pl.ds(g * 16, 16)] = raw >> 3
                off8_v[t, pl.ds(g * 16, 16)] = raw & 7

        @pl.loop(0, L)
        def _sph(k):
            @pl.loop(0, C // 16)
            def _sphg(g):
                raw = histp_v[k, pl.ds(g * 16, 16)]
                hhi_v[k, pl.ds(g * 16, 16)] = raw >> 3
                hoff_v[k, pl.ds(g * 16, 16)] = raw & 7

        fire(0, tables[0], 0)
        fire(1, tables[1], 1)
        for p in range(NE + 1):
            b = p % 2
            drain(b)
            select(p, p * D if p < NE else HIST_ROW, False, b)
            if p + 2 < NE:
                fire(p + 2, tables[p + 2], b)
            else:
                fire(p + 2, ehT, b)

        @pl.loop(NE + 1, NE + L - 1, step=2)
        def _hist(p):
            drain(1)
            select(p, HIST_ROW, True, 1)

            @pl.when(p + 2 < NE + L)
            def _f1():
                fire(p + 2, ehT, 1)

            drain(0)
            select(p + 1, HIST_ROW, True, 0)

            @pl.when(p + 3 < NE + L)
            def _f0():
                fire(p + 3, ehT, 0)

        drain(1)
        select(NE + L - 1, HIST_ROW, True, 1)

        pltpu.sync_copy(featsT_v, featsT_hbm.at[:, pl.ds(base, C)])


def _sc_gather(idx8, histT, tT, ehT, dummy):
    mesh = plsc.VectorSubcoreMesh(core_axis_name="c", subcore_axis_name="s")
    return pl.kernel(
        _sc_gather_body,
        out_type=jax.ShapeDtypeStruct((F, B), jnp.float32),
        mesh=mesh,
        scratch_types=[
            pltpu.VMEM((NE, C), jnp.int32),
            pltpu.VMEM((LP, C), jnp.int32),
            pltpu.VMEM((NE, C), jnp.int32),
            pltpu.VMEM((NE, C), jnp.int32),
            pltpu.VMEM((LP, C), jnp.int32),
            pltpu.VMEM((LP, C), jnp.int32),
            pltpu.VMEM((2, D, C, 8), jnp.float32),
            pltpu.VMEM((F, C), jnp.float32),
            pltpu.SemaphoreType.DMA,
            pltpu.SemaphoreType.DMA,
        ],
        compiler_params=pltpu.CompilerParams(use_tc_tiling_on_sc=False,
                                             needs_layout_passes=False),
    )(idx8, histT, *tT, ehT, dummy)


def _mlp_body(x_ref, ctn_ref, wv_ref, W1_ref, b1_ref, W2_ref, b2_ref,
              W3_ref, b3_ref, o_ref):
    xt = x_ref[...]
    h = lax.dot_general(xt, W1_ref[...], (((0,), (0,)), ((), ())))
    h = jnp.maximum(h + b1_ref[...][None, :], 0.0)
    h = jnp.maximum(h @ W2_ref[...] + b2_ref[...][None, :], 0.0)
    z = h @ W3_ref[...]
    lin = ctn_ref[...] @ wv_ref[...]
    r = z[:, 0] + lin[:, 0] + b3_ref[0]
    o_ref[...] = jax.nn.sigmoid(r)


def _mlp(featsT, ctn, wvec, W1, b1, W2, b2, W3, b3):
    bm = 2048
    grid = (B // bm,)
    return pl.pallas_call(
        _mlp_body,
        grid=grid,
        in_specs=[
            pl.BlockSpec((F, bm), lambda i: (0, i)),
            pl.BlockSpec((bm, 4), lambda i: (i, 0)),
            pl.BlockSpec((4, 1), lambda i: (0, 0)),
            pl.BlockSpec((F, 256), lambda i: (0, 0)),
            pl.BlockSpec((256,), lambda i: (0,)),
            pl.BlockSpec((256, 128), lambda i: (0, 0)),
            pl.BlockSpec((128,), lambda i: (0,)),
            pl.BlockSpec((128, 1), lambda i: (0, 0)),
            pl.BlockSpec((1,), lambda i: (0,)),
        ],
        out_specs=pl.BlockSpec((bm,), lambda i: (i,)),
        out_shape=jax.ShapeDtypeStruct((B,), jnp.float32),
    )(featsT, ctn, wvec, W1, b1, W2, b2, W3, b3)


def kernel(user_id, item_id, cat_0, cat_1, cat_2, cat_3, cat_4, cat_5,
           ctn_0, ctn_1, ctn_2, ctn_3, hist_item,
           emb_user, emb_item, emb_cat_0, emb_cat_1, emb_cat_2, emb_cat_3,
           emb_cat_4, emb_cat_5, emb_hist,
           w_ctn_0, w_ctn_1, w_ctn_2, w_ctn_3,
           W1, b1, W2, b2, W3, b3):
    idx8 = jnp.stack([
        user_id[:, 0], item_id[:, 0], cat_0[:, 0], cat_1[:, 0],
        cat_2[:, 0], cat_3[:, 0], cat_4[:, 0], cat_5[:, 0],
    ]).astype(jnp.int32)
    histT = hist_item.T.astype(jnp.int32)
    histT = jnp.pad(histT, ((0, LP - L), (0, 0)))
    dummy = jnp.zeros((D, C, 8), jnp.float32)

    pv = lambda t: t.T.reshape(D, t.shape[0] // 8, 8)
    tT = [pv(t) for t in (emb_user, emb_item, emb_cat_0, emb_cat_1,
                          emb_cat_2, emb_cat_3, emb_cat_4, emb_cat_5)]
    featsT = _sc_gather(idx8, histT, tT, pv(emb_hist), dummy)

    ctn = jnp.concatenate([ctn_0, ctn_1, ctn_2, ctn_3], axis=1)
    wvec = jnp.stack([w_ctn_0[0, 0], w_ctn_1[0, 0], w_ctn_2[0, 0],
                      w_ctn_3[0, 0]]).reshape(4, 1)
    return _mlp(featsT, ctn, wvec, W1, b1, W2, b2, W3, b3)


# speedup vs baseline: 3.3712x; 3.3712x over previous
import jax
import jax.numpy as jnp
from jax import lax
from jax.experimental import pallas as pl
from jax.experimental.pallas import tpu as pltpu
from jax.experimental.pallas import tpu_sc as plsc

B = 16384
D = 16
L = 50
NE = 8
F = (NE + 1) * D
NC = 2
NS = 16
NW = NC * NS
S = B // NW
C = 128
NCH = S // C

HIST_COL = NE * D


def _sc_gather_body(idx8_hbm, hist_hbm,
                    emb_user, emb_item, ec0, ec1, ec2, ec3, ec4, ec5,
                    emb_hist,
                    feats_hbm,
                    idx8_v, hidx_v, rows8_v, hrows_v, feats_v,
                    sem_g, sem_h0, sem_h1):
    tables = (emb_user, emb_item, ec0, ec1, ec2, ec3, ec4, ec5)
    wid = lax.axis_index("s") * NC + lax.axis_index("c")

    @pl.loop(0, NCH)
    def _chunk(c):
        base = wid * S + c * C

        pltpu.sync_copy(idx8_hbm.at[:, pl.ds(base, C)], idx8_v)
        pltpu.sync_copy(hist_hbm.at[:, pl.ds(base, C)], hidx_v)

        descs = []
        for t in range(NE):
            descs.append(pltpu.async_copy(
                tables[t].at[idx8_v.at[t]], rows8_v.at[t], sem_g))

        sems = (sem_h0, sem_h1)

        def hfire(k, b):
            pltpu.async_copy(emb_hist.at[hidx_v.at[k]],
                             hrows_v.at[b], sems[b])

        def hwait(b):
            pltpu.make_async_copy(emb_hist.at[hidx_v.at[0]],
                                  hrows_v.at[b], sems[b]).wait()

        def hacc(b, store):
            @pl.loop(0, C, step=16)
            def _acc(r0):
                for dr in range(16):
                    r = r0 + dr
                    if store:
                        feats_v[r, pl.ds(HIST_COL, D)] = hrows_v[b, r, :]
                    else:
                        plsc.addupdate(feats_v.at[r, pl.ds(HIST_COL, D)],
                                       hrows_v[b, r, :])

        hfire(0, 0)
        hfire(1, 1)
        hwait(0)
        hacc(0, True)
        hfire(2, 0)

        @pl.loop(1, L - 1, step=2)
        def _hist(k):
            hwait(1)
            hacc(1, False)

            @pl.when(k + 2 < L)
            def _f1():
                hfire(k + 2, 1)

            hwait(0)
            hacc(0, False)

            @pl.when(k + 3 < L)
            def _f0():
                hfire(k + 3, 0)

        hwait(1)
        hacc(1, False)

        for t in range(NE):
            descs[t].wait()
        for t in range(NE):
            @pl.loop(0, C, step=16)
            def _place(r0, t=t):
                for dr in range(16):
                    r = r0 + dr
                    feats_v[r, pl.ds(t * D, D)] = rows8_v[t, r, :]

        pltpu.sync_copy(feats_v, feats_hbm.at[pl.ds(base, C), :])


def _sc_gather(idx8, histT, emb_user, emb_item, ec0, ec1, ec2, ec3, ec4,
               ec5, emb_hist):
    mesh = plsc.VectorSubcoreMesh(core_axis_name="c", subcore_axis_name="s")
    return pl.kernel(
        _sc_gather_body,
        out_type=jax.ShapeDtypeStruct((B, F), jnp.float32),
        mesh=mesh,
        scratch_types=[
            pltpu.VMEM((NE, C), jnp.int32),
            pltpu.VMEM((L, C), jnp.int32),
            pltpu.VMEM((NE, C, D), jnp.float32),
            pltpu.VMEM((2, C, D), jnp.float32),
            pltpu.VMEM((C, F), jnp.float32),
            pltpu.SemaphoreType.DMA,
            pltpu.SemaphoreType.DMA,
            pltpu.SemaphoreType.DMA,
        ],
        compiler_params=pltpu.CompilerParams(use_tc_tiling_on_sc=False),
    )(idx8, histT, emb_user, emb_item, ec0, ec1, ec2, ec3, ec4, ec5,
      emb_hist)


def _mlp_body(x_ref, ctn_ref, wv_ref, W1_ref, b1_ref, W2_ref, b2_ref,
              W3_ref, b3_ref, o_ref):
    x = x_ref[...]
    h = jnp.maximum(x @ W1_ref[...] + b1_ref[...][None, :], 0.0)
    h = jnp.maximum(h @ W2_ref[...] + b2_ref[...][None, :], 0.0)
    z = h @ W3_ref[...]
    lin = ctn_ref[...] @ wv_ref[...]
    r = z[:, 0] + lin[:, 0] + b3_ref[0]
    o_ref[...] = jax.nn.sigmoid(r)


def _mlp(feats, ctn, wvec, W1, b1, W2, b2, W3, b3):
    bm = 2048
    grid = (B // bm,)
    return pl.pallas_call(
        _mlp_body,
        grid=grid,
        in_specs=[
            pl.BlockSpec((bm, F), lambda i: (i, 0)),
            pl.BlockSpec((bm, 4), lambda i: (i, 0)),
            pl.BlockSpec((4, 1), lambda i: (0, 0)),
            pl.BlockSpec((F, 256), lambda i: (0, 0)),
            pl.BlockSpec((256,), lambda i: (0,)),
            pl.BlockSpec((256, 128), lambda i: (0, 0)),
            pl.BlockSpec((128,), lambda i: (0,)),
            pl.BlockSpec((128, 1), lambda i: (0, 0)),
            pl.BlockSpec((1,), lambda i: (0,)),
        ],
        out_specs=pl.BlockSpec((bm,), lambda i: (i,)),
        out_shape=jax.ShapeDtypeStruct((B,), jnp.float32),
    )(feats, ctn, wvec, W1, b1, W2, b2, W3, b3)


def kernel(user_id, item_id, cat_0, cat_1, cat_2, cat_3, cat_4, cat_5,
           ctn_0, ctn_1, ctn_2, ctn_3, hist_item,
           emb_user, emb_item, emb_cat_0, emb_cat_1, emb_cat_2, emb_cat_3,
           emb_cat_4, emb_cat_5, emb_hist,
           w_ctn_0, w_ctn_1, w_ctn_2, w_ctn_3,
           W1, b1, W2, b2, W3, b3):
    idx8 = jnp.stack([
        user_id[:, 0], item_id[:, 0], cat_0[:, 0], cat_1[:, 0],
        cat_2[:, 0], cat_3[:, 0], cat_4[:, 0], cat_5[:, 0],
    ]).astype(jnp.int32)
    histT = hist_item.T.astype(jnp.int32)

    feats = _sc_gather(idx8, histT, emb_user, emb_item, emb_cat_0,
                       emb_cat_1, emb_cat_2, emb_cat_3, emb_cat_4,
                       emb_cat_5, emb_hist)

    ctn = jnp.concatenate([ctn_0, ctn_1, ctn_2, ctn_3], axis=1)
    wvec = jnp.stack([w_ctn_0[0, 0], w_ctn_1[0, 0], w_ctn_2[0, 0],
                      w_ctn_3[0, 0]]).reshape(4, 1)
    return _mlp(feats, ctn, wvec, W1, b1, W2, b2, W3, b3)
